# baseline (device time: 57098 ns/iter reference)
import jax
import jax.numpy as jnp
from jax import lax
from jax.experimental import pallas as pl
from jax.experimental.pallas import tpu as pltpu

N_DEV = 4
B = 2
S = 1024
SP = S // N_DEV
D = 768
H = 4
DH = 64
DHA = 128
R = B * S
NP = 2
SPL = SP // NP


def kernel(x, Wq, Wk, Wv, Wo):
    def body(x_ref, wq_ref, wk_ref, wv_ref, wo_ref, out_ref,
             xfull, q_ref, k_ref, v_ref, ctx_ref, partial,
             tbuf, dbuf, cbuf, stage,
             ag_ssem, ag_rsem, t_ssem, t_rsem, d_ssem, d_rsem,
             c_ssem, c_rsem):
        my = lax.axis_index("i")
        left = (my - 1) % N_DEV
        right = (my + 1) % N_DEV
        opp = (my + 2) % N_DEV

        xfull[:, pl.ds(my * SP, SP), :] = x_ref[...].astype(jnp.bfloat16)

        barrier_sem = pltpu.get_barrier_semaphore()
        for nbr in (left, right, opp):
            pl.semaphore_signal(
                barrier_sem, inc=1,
                device_id=(nbr,), device_id_type=pl.DeviceIdType.MESH,
            )
        pl.semaphore_wait(barrier_sem, 3)

        ag_sends = []
        for idx, tgt in enumerate((right, left, opp)):
            for p in range(NP):
                r = pltpu.make_async_remote_copy(
                    src_ref=xfull.at[:, pl.ds(my * SP + p * SPL, SPL), :],
                    dst_ref=xfull.at[:, pl.ds(my * SP + p * SPL, SPL), :],
                    send_sem=ag_ssem.at[idx, p],
                    recv_sem=ag_rsem.at[idx, p],
                    device_id=(tgt,),
                    device_id_type=pl.DeviceIdType.MESH,
                )
                r.start()
                ag_sends.append(r)

        RCP = B * SPL
        lane = lax.broadcasted_iota(jnp.int32, (RCP, H * DH), 1)
        srow = (lax.broadcasted_iota(jnp.int32, (RCP, H * DH), 0) % SPL
                ).astype(jnp.float32)
        j = ((lane % DH) // 2).astype(jnp.float32)
        inv = jnp.exp(-jnp.log(10000.0) * (2.0 * j) / DH)
        even = (lane % 2) == 0
        wq_bf = wq_ref[...].astype(jnp.bfloat16)
        wk_bf = wk_ref[...].astype(jnp.bfloat16)
        wv_bf = wv_ref[...].astype(jnp.bfloat16)
        ones_col = jnp.full((RCP, 1), 1.0, jnp.bfloat16)
        zeros_pad = jnp.zeros((RCP, DHA - DH - 1), jnp.bfloat16)

        def qkv_piece(c, p):
            base = c * SP + p * SPL
            ang = (srow + base.astype(jnp.float32)) * inv
            cos_t = jnp.cos(ang)
            sin_t = jnp.sin(ang)

            def rope(t):
                tm1 = jnp.concatenate([t[:, 1:], t[:, :1]], axis=-1)
                tp1 = jnp.concatenate([t[:, -1:], t[:, :-1]], axis=-1)
                t_r = jnp.where(even, -tm1, tp1)
                return t * cos_t + t_r * sin_t

            xc = xfull[:, pl.ds(base, SPL), :].reshape(RCP, D)
            q_c = (rope(jnp.dot(xc, wq_bf,
                                preferred_element_type=jnp.float32))
                   * 0.125).astype(jnp.bfloat16)
            k_c = rope(jnp.dot(xc, wk_bf,
                               preferred_element_type=jnp.float32)
                       ).astype(jnp.bfloat16)
            v_c = jnp.dot(xc, wv_bf,
                          preferred_element_type=jnp.float32)
            for h in range(H):
                cols = slice(h * DH, (h + 1) * DH)
                va = jnp.concatenate(
                    [v_c[:, cols].astype(jnp.bfloat16), ones_col, zeros_pad],
                    axis=-1)
                for b in range(B):
                    brows = slice(b * SPL, (b + 1) * SPL)
                    rows = pl.ds(b * S + base, SPL)
                    q_ref[h, rows, :] = q_c[brows, cols]
                    k_ref[h, rows, :] = k_c[brows, cols]
                    v_ref[h, rows, :] = va[brows, :]

        for p in range(NP):
            qkv_piece(my, p)
        for p in range(NP):
            for idx, src_pos in enumerate((left, right, opp)):
                rd = pltpu.make_async_remote_copy(
                    src_ref=xfull.at[:, pl.ds(my * SP + p * SPL, SPL), :],
                    dst_ref=xfull.at[:, pl.ds(src_pos * SP + p * SPL, SPL), :],
                    send_sem=ag_ssem.at[idx, p],
                    recv_sem=ag_rsem.at[idx, p],
                    device_id=(src_pos,),
                    device_id_type=pl.DeviceIdType.MESH,
                )
                rd.wait_recv()
                qkv_piece(src_pos, p)
        for r in ag_sends:
            r.wait_send()

        wo_bf = wo_ref[...].astype(jnp.bfloat16)
        rs_sends = []

        def chunk_partial(tgt):
            for b in range(B):
                rows_full = slice(b * S, (b + 1) * S)

                def attn_h(h, _, rows_full=rows_full, b=b):
                    qc = q_ref[h, pl.ds(b * S + tgt * SP, SP), :]
                    kb = k_ref[h, rows_full, :]
                    s = lax.dot_general(
                        qc, kb, (((1,), (1,)), ((), ())),
                        preferred_element_type=jnp.float32,
                    )
                    w = jnp.exp(s.astype(jnp.bfloat16))
                    ca = jnp.dot(w, v_ref[h, rows_full, :],
                                 preferred_element_type=jnp.float32)
                    ctx_ref[h, pl.ds(b * SP, SP), :] = (
                        ca[:, :DH] / ca[:, DH:DH + 1]).astype(jnp.bfloat16)
                    return _

                lax.fori_loop(0, H, attn_h, None)
            co = jnp.concatenate(
                [ctx_ref[h] for h in range(H)], axis=-1)
            return jnp.dot(co, wo_bf, preferred_element_type=jnp.float32)

        def _send(src, tgt, dst, ssem, rsem):
            r = pltpu.make_async_remote_copy(
                src_ref=src, dst_ref=dst, send_sem=ssem, recv_sem=rsem,
                device_id=(tgt,), device_id_type=pl.DeviceIdType.MESH,
            )
            r.start()
            rs_sends.append(r)

        def _wait_recv(dst, rsem):
            pltpu.make_async_remote_copy(
                src_ref=stage.at[0], dst_ref=dst,
                send_sem=t_ssem.at[0], recv_sem=rsem,
                device_id=(my,), device_id_type=pl.DeviceIdType.MESH,
            ).wait_recv()

        pc = chunk_partial(opp).astype(jnp.bfloat16).reshape(B, SP, D)
        partial[:, pl.ds(opp * SP, SP), :] = pc
        _send(partial.at[0, pl.ds(opp * SP, SP), :], right, tbuf.at[0],
              t_ssem.at[0], t_rsem.at[0])
        _send(partial.at[1, pl.ds(opp * SP, SP), :], left, tbuf.at[1],
              t_ssem.at[1], t_rsem.at[1])

        pc = chunk_partial(right).astype(jnp.bfloat16).reshape(B, SP, D)
        partial[:, pl.ds(right * SP, SP), :] = pc
        _send(partial.at[1, pl.ds(right * SP, SP), :], right, dbuf.at[0],
              d_ssem.at[0], d_rsem.at[0])

        pc = chunk_partial(left).astype(jnp.bfloat16).reshape(B, SP, D)
        partial[:, pl.ds(left * SP, SP), :] = pc
        _send(partial.at[0, pl.ds(left * SP, SP), :], left, dbuf.at[1],
              d_ssem.at[1], d_rsem.at[1])

        _wait_recv(tbuf.at[0], t_rsem.at[0])
        stage[0] = partial[0, pl.ds(right * SP, SP), :] + tbuf[0]
        _send(stage.at[0], right, cbuf.at[0], c_ssem.at[0], c_rsem.at[0])
        _wait_recv(tbuf.at[1], t_rsem.at[1])
        stage[1] = partial[1, pl.ds(left * SP, SP), :] + tbuf[1]
        _send(stage.at[1], left, cbuf.at[1], c_ssem.at[1], c_rsem.at[1])

        mine = chunk_partial(my).reshape(B, SP, D)

        _wait_recv(cbuf.at[0], c_rsem.at[0])
        _wait_recv(cbuf.at[1], c_rsem.at[1])
        _wait_recv(dbuf.at[0], d_rsem.at[0])
        _wait_recv(dbuf.at[1], d_rsem.at[1])
        out_ref[0] = (mine[0] + cbuf[0].astype(jnp.float32)
                      + dbuf[1].astype(jnp.float32))
        out_ref[1] = (mine[1] + cbuf[1].astype(jnp.float32)
                      + dbuf[0].astype(jnp.float32))
        for r in rs_sends:
            r.wait_send()

    return pl.pallas_call(
        body,
        out_shape=jax.ShapeDtypeStruct((B, SP, D), jnp.float32),
        in_specs=[pl.BlockSpec(memory_space=pltpu.VMEM)] * 5,
        out_specs=pl.BlockSpec(memory_space=pltpu.VMEM),
        scratch_shapes=[
            pltpu.VMEM((B, S, D), jnp.bfloat16),
            pltpu.VMEM((H, R, DH), jnp.bfloat16),
            pltpu.VMEM((H, R, DH), jnp.bfloat16),
            pltpu.VMEM((H, R, DHA), jnp.bfloat16),
            pltpu.VMEM((H, B * SP, DH), jnp.bfloat16),
            pltpu.VMEM((B, S, D), jnp.bfloat16),
            pltpu.VMEM((2, SP, D), jnp.bfloat16),
            pltpu.VMEM((2, SP, D), jnp.bfloat16),
            pltpu.VMEM((2, SP, D), jnp.bfloat16),
            pltpu.VMEM((2, SP, D), jnp.bfloat16),
            pltpu.SemaphoreType.DMA((3, NP)),
            pltpu.SemaphoreType.DMA((3, NP)),
            pltpu.SemaphoreType.DMA((2,)),
            pltpu.SemaphoreType.DMA((2,)),
            pltpu.SemaphoreType.DMA((2,)),
            pltpu.SemaphoreType.DMA((2,)),
            pltpu.SemaphoreType.DMA((2,)),
            pltpu.SemaphoreType.DMA((2,)),
        ],
        compiler_params=pltpu.CompilerParams(
            collective_id=0,
            vmem_limit_bytes=60 * 1024 * 1024,
        ),
    )(x, Wq, Wk, Wv, Wo)
